# baseline (speedup 1.0000x reference)
import jax
import jax.numpy as jnp
from jax.experimental import pallas as pl
from jax.experimental.pallas import tpu as pltpu

N_INPUT = 1000
N_HIDDEN = 64
N_LATENT = 32
B = 1024


def _prologue_body(m_ref, x_ref, xo_ref):
    m2 = m_ref[...]
    cols = jax.lax.broadcasted_iota(jnp.int32, (1, N_INPUT), 1)
    hit = jnp.any(m2 == cols, axis=0, keepdims=True)
    colmask = jnp.where(hit, 0.0, 1.0)
    xo_ref[...] = jnp.log1p(x_ref[...] * colmask)


def _matvec_body(m_sref, a_ref, x_ref, b_ref, h_ref):
    a = a_ref[0]
    xv = x_ref[0]
    h = jax.lax.dot_general(xv, a, (((1,), (1,)), ((), ())),
                            preferred_element_type=jnp.float32)
    h_ref[0] = h + b_ref[0]


def _epilogue_body(h_ref, w1_ref, b1_ref, s_ref, lb_ref, w3_ref, b3_ref,
                   w4_ref, b4_ref, mean_ref, scale_ref):
    h = h_ref[...]
    h = jnp.dot(h, w1_ref[...], preferred_element_type=jnp.float32) + b1_ref[...]
    mu = jnp.mean(h, axis=-1, keepdims=True)
    var = jnp.mean((h - mu) ** 2, axis=-1, keepdims=True)
    h = (h - mu) * jax.lax.rsqrt(var + 1e-6) * s_ref[...] + lb_ref[...]
    h = jnp.maximum(h, 0.0)
    mean_ref[...] = jnp.dot(h, w3_ref[...], preferred_element_type=jnp.float32) + b3_ref[...]
    scale_ref[...] = jnp.exp(
        jnp.dot(h, w4_ref[...], preferred_element_type=jnp.float32) + b4_ref[...])


def kernel(x, masked_genes, amats_table, bvecs_table, W1, b1, ln_scale, ln_bias,
           W3, b3, W4, b4):
    m = masked_genes.astype(jnp.int32)

    x_ = pl.pallas_call(
        _prologue_body,
        out_shape=jax.ShapeDtypeStruct((B, N_INPUT), jnp.float32),
    )(m.reshape(B, 1), x)

    amats3 = amats_table.reshape(N_INPUT, N_HIDDEN, N_INPUT)
    bvecs3 = bvecs_table.reshape(N_INPUT, 1, N_HIDDEN)
    x3 = x_.reshape(B, 1, N_INPUT)

    h = pl.pallas_call(
        _matvec_body,
        grid_spec=pltpu.PrefetchScalarGridSpec(
            num_scalar_prefetch=1,
            grid=(B,),
            in_specs=[
                pl.BlockSpec((1, N_HIDDEN, N_INPUT), lambda i, m: (m[i], 0, 0)),
                pl.BlockSpec((1, 1, N_INPUT), lambda i, m: (i, 0, 0)),
                pl.BlockSpec((1, 1, N_HIDDEN), lambda i, m: (m[i], 0, 0)),
            ],
            out_specs=pl.BlockSpec((1, 1, N_HIDDEN), lambda i, m: (i, 0, 0)),
        ),
        out_shape=jax.ShapeDtypeStruct((B, 1, N_HIDDEN), jnp.float32),
    )(m, amats3, x3, bvecs3)

    mean, scale = pl.pallas_call(
        _epilogue_body,
        out_shape=(
            jax.ShapeDtypeStruct((B, N_LATENT), jnp.float32),
            jax.ShapeDtypeStruct((B, N_LATENT), jnp.float32),
        ),
    )(h.reshape(B, N_HIDDEN), W1, b1.reshape(1, N_HIDDEN),
      ln_scale.reshape(1, N_HIDDEN), ln_bias.reshape(1, N_HIDDEN),
      W3, b3.reshape(1, N_LATENT), W4, b4.reshape(1, N_LATENT))
    return (mean, scale)


# speedup vs baseline: 2.0324x; 2.0324x over previous
import jax
import jax.numpy as jnp
from jax.experimental import pallas as pl
from jax.experimental.pallas import tpu as pltpu

N_INPUT = 1000
N_HIDDEN = 64
N_LATENT = 32
B = 1024


def _prologue_body(m_ref, x_ref, xo_ref):
    m2 = m_ref[...]
    cols = jax.lax.broadcasted_iota(jnp.int32, (1, N_INPUT), 1)
    hit = jnp.any(m2 == cols, axis=0, keepdims=True)
    colmask = jnp.where(hit, 0.0, 1.0)
    xo_ref[...] = jnp.log1p(x_ref[...] * colmask)


ROWS_PER_STEP = 8


def _matvec_body(m_sref, *refs):
    a_refs = refs[:ROWS_PER_STEP]
    x_ref = refs[ROWS_PER_STEP]
    b_refs = refs[ROWS_PER_STEP + 1:2 * ROWS_PER_STEP + 1]
    h_ref = refs[2 * ROWS_PER_STEP + 1]
    a8 = jnp.concatenate([a[0] for a in a_refs], axis=0)
    xs = x_ref[0]
    res = jax.lax.dot_general(xs, a8, (((1,), (1,)), ((), ())),
                              preferred_element_type=jnp.float32)
    e_iota = jax.lax.broadcasted_iota(jnp.int32, (ROWS_PER_STEP, ROWS_PER_STEP * N_HIDDEN), 0)
    r_iota = jax.lax.broadcasted_iota(jnp.int32, (ROWS_PER_STEP, ROWS_PER_STEP * N_HIDDEN), 1)
    mask = (r_iota // N_HIDDEN) == e_iota
    h2 = jnp.sum(jnp.where(mask, res, 0.0), axis=0, keepdims=True)
    bv = jnp.concatenate([b[0] for b in b_refs], axis=1)
    h_ref[0] = h2 + bv


def _epilogue_body(h_ref, w1_ref, b1_ref, s_ref, lb_ref, w3_ref, b3_ref,
                   w4_ref, b4_ref, mean_ref, scale_ref):
    h = h_ref[...]
    h = jnp.dot(h, w1_ref[...], preferred_element_type=jnp.float32) + b1_ref[...]
    mu = jnp.mean(h, axis=-1, keepdims=True)
    var = jnp.mean((h - mu) ** 2, axis=-1, keepdims=True)
    h = (h - mu) * jax.lax.rsqrt(var + 1e-6) * s_ref[...] + lb_ref[...]
    h = jnp.maximum(h, 0.0)
    mean_ref[...] = jnp.dot(h, w3_ref[...], preferred_element_type=jnp.float32) + b3_ref[...]
    scale_ref[...] = jnp.exp(
        jnp.dot(h, w4_ref[...], preferred_element_type=jnp.float32) + b4_ref[...])


def kernel(x, masked_genes, amats_table, bvecs_table, W1, b1, ln_scale, ln_bias,
           W3, b3, W4, b4):
    m = masked_genes.astype(jnp.int32)

    x_ = pl.pallas_call(
        _prologue_body,
        out_shape=jax.ShapeDtypeStruct((B, N_INPUT), jnp.float32),
    )(m.reshape(B, 1), x)

    amats3 = amats_table.reshape(N_INPUT, N_HIDDEN, N_INPUT)
    bvecs3 = bvecs_table.reshape(N_INPUT, 1, N_HIDDEN)
    nsteps = B // ROWS_PER_STEP
    x3 = x_.reshape(nsteps, ROWS_PER_STEP, N_INPUT)

    def _a_map(j):
        return lambda i, m: (m[ROWS_PER_STEP * i + j], 0, 0)

    a_specs = [pl.BlockSpec((1, N_HIDDEN, N_INPUT), _a_map(j))
               for j in range(ROWS_PER_STEP)]
    b_specs = [pl.BlockSpec((1, 1, N_HIDDEN), _a_map(j))
               for j in range(ROWS_PER_STEP)]

    h = pl.pallas_call(
        _matvec_body,
        grid_spec=pltpu.PrefetchScalarGridSpec(
            num_scalar_prefetch=1,
            grid=(nsteps,),
            in_specs=a_specs
            + [pl.BlockSpec((1, ROWS_PER_STEP, N_INPUT), lambda i, m: (i, 0, 0))]
            + b_specs,
            out_specs=pl.BlockSpec((1, 1, ROWS_PER_STEP * N_HIDDEN),
                                   lambda i, m: (i, 0, 0)),
        ),
        out_shape=jax.ShapeDtypeStruct((nsteps, 1, ROWS_PER_STEP * N_HIDDEN),
                                       jnp.float32),
    )(m, *([amats3] * ROWS_PER_STEP), x3, *([bvecs3] * ROWS_PER_STEP))

    mean, scale = pl.pallas_call(
        _epilogue_body,
        out_shape=(
            jax.ShapeDtypeStruct((B, N_LATENT), jnp.float32),
            jax.ShapeDtypeStruct((B, N_LATENT), jnp.float32),
        ),
    )(h.reshape(B, N_HIDDEN), W1, b1.reshape(1, N_HIDDEN),
      ln_scale.reshape(1, N_HIDDEN), ln_bias.reshape(1, N_HIDDEN),
      W3, b3.reshape(1, N_LATENT), W4, b4.reshape(1, N_LATENT))
    return (mean, scale)


# speedup vs baseline: 2.0739x; 1.0204x over previous
import functools

import jax
import jax.numpy as jnp
from jax import lax
from jax.experimental import pallas as pl
from jax.experimental.pallas import tpu as pltpu
from jax.experimental.pallas import tpu_sc as plsc

N_INPUT = 1000
N_HIDDEN = 64
N_LATENT = 32
B = 1024

NC = 2
NS = 16
L = 16
NW = NC * NS
BPW = B // NW
NFULL = N_INPUT // L
TAIL_OFF = N_INPUT - L


def _prologue_body(m_ref, x_ref, xo_ref):
    m2 = m_ref[...]
    cols = lax.broadcasted_iota(jnp.int32, (1, N_INPUT), 1)
    hit = jnp.any(m2 == cols, axis=0, keepdims=True)
    colmask = jnp.where(hit, 0.0, 1.0)
    xo_ref[...] = jnp.log1p(x_ref[...] * colmask)


def _sc_matvec_body(x_hbm, m_hbm, tab_hbm, h_hbm,
                    m_v, idx_v, x_v, a_v, h_v, hstage, sem_a):
    wid = lax.axis_index("s") * NC + lax.axis_index("c")
    base = wid * BPW
    pltpu.sync_copy(m_hbm.at[pl.ds(base, BPW)], m_v)
    lane = lax.iota(jnp.int32, L)
    tail_mask = (lane >= L - N_INPUT % L).astype(jnp.float32)

    def example_body(i, carry):
        m16 = plsc.load_gather(m_v, [jnp.full((L,), i, jnp.int32)])
        row0 = m16 * N_HIDDEN
        for c in range(N_HIDDEN // L):
            idx_v[pl.ds(L * c, L)] = row0 + (L * c) + lane
        pltpu.sync_copy(x_hbm.at[base + i], x_v)
        pltpu.async_copy(tab_hbm.at[idx_v], a_v, sem_a).wait()
        for g in range(N_HIDDEN // L):
            def kbody(k, accs):
                off = pl.multiple_of(L * k, L)
                xk = x_v[pl.ds(off, L)]
                return tuple(accs[j] + a_v[L * g + j, pl.ds(off, L)] * xk
                             for j in range(L))
            accs = lax.fori_loop(
                0, NFULL, kbody,
                tuple(jnp.zeros((L,), jnp.float32) for _ in range(L)))
            xt = x_v[pl.ds(TAIL_OFF, L)] * tail_mask
            for j in range(L):
                hstage[pl.ds(L * j, L)] = (
                    accs[j] + a_v[L * g + j, pl.ds(TAIL_OFF, L)] * xt)
            hvec = jnp.zeros((L,), jnp.float32)
            for c in range(L):
                hvec = hvec + plsc.load_gather(hstage, [lane * L + c])
            off_h = pl.multiple_of(i * N_HIDDEN + L * g, L)
            h_v[pl.ds(off_h, L)] = hvec
        return carry

    lax.fori_loop(0, BPW, example_body, 0)
    pltpu.sync_copy(h_v, h_hbm.at[pl.ds(base * N_HIDDEN, BPW * N_HIDDEN)])


def _epilogue_body(h_ref, m_ref, bvt_ref, w1_ref, b1_ref, s_ref, lb_ref,
                   w3_ref, b3_ref, w4_ref, b4_ref, mean_ref, scale_ref):
    cols = lax.broadcasted_iota(jnp.int32, (B, N_INPUT), 1)
    onehot = (m_ref[...] == cols).astype(jnp.float32)
    h = h_ref[...] + jnp.dot(onehot, bvt_ref[...],
                             preferred_element_type=jnp.float32)
    h = jnp.dot(h, w1_ref[...], preferred_element_type=jnp.float32) + b1_ref[...]
    mu = jnp.mean(h, axis=-1, keepdims=True)
    var = jnp.mean((h - mu) ** 2, axis=-1, keepdims=True)
    h = (h - mu) * lax.rsqrt(var + 1e-6) * s_ref[...] + lb_ref[...]
    h = jnp.maximum(h, 0.0)
    mean_ref[...] = jnp.dot(h, w3_ref[...], preferred_element_type=jnp.float32) + b3_ref[...]
    scale_ref[...] = jnp.exp(
        jnp.dot(h, w4_ref[...], preferred_element_type=jnp.float32) + b4_ref[...])


def kernel(x, masked_genes, amats_table, bvecs_table, W1, b1, ln_scale, ln_bias,
           W3, b3, W4, b4):
    m = masked_genes.astype(jnp.int32)

    x_ = pl.pallas_call(
        _prologue_body,
        out_shape=jax.ShapeDtypeStruct((B, N_INPUT), jnp.float32),
    )(m.reshape(B, 1), x)

    tab2 = amats_table.reshape(N_INPUT * N_HIDDEN, N_INPUT)

    sc = pl.kernel(
        _sc_matvec_body,
        out_type=jax.ShapeDtypeStruct((B * N_HIDDEN,), jnp.float32),
        mesh=plsc.VectorSubcoreMesh(core_axis_name="c", subcore_axis_name="s"),
        compiler_params=pltpu.CompilerParams(needs_layout_passes=False,
                                             use_tc_tiling_on_sc=False),
        scratch_types=[
            pltpu.VMEM((BPW,), jnp.int32),
            pltpu.VMEM((N_HIDDEN,), jnp.int32),
            pltpu.VMEM((N_INPUT,), jnp.float32),
            pltpu.VMEM((N_HIDDEN, N_INPUT), jnp.float32),
            pltpu.VMEM((BPW * N_HIDDEN,), jnp.float32),
            pltpu.VMEM((L * L,), jnp.float32),
            pltpu.SemaphoreType.DMA,
        ],
    )
    h = sc(x_, m, tab2)

    mean, scale = pl.pallas_call(
        _epilogue_body,
        out_shape=(
            jax.ShapeDtypeStruct((B, N_LATENT), jnp.float32),
            jax.ShapeDtypeStruct((B, N_LATENT), jnp.float32),
        ),
    )(h.reshape(B, N_HIDDEN), m.reshape(B, 1), bvecs_table,
      W1, b1.reshape(1, N_HIDDEN),
      ln_scale.reshape(1, N_HIDDEN), ln_bias.reshape(1, N_HIDDEN),
      W3, b3.reshape(1, N_LATENT), W4, b4.reshape(1, N_LATENT))
    return (mean, scale)


# speedup vs baseline: 2.6464x; 1.2760x over previous
import functools

import jax
import jax.numpy as jnp
from jax import lax
from jax.experimental import pallas as pl
from jax.experimental.pallas import tpu as pltpu
from jax.experimental.pallas import tpu_sc as plsc

N_INPUT = 1000
N_HIDDEN = 64
N_LATENT = 32
B = 1024

NC = 2
NS = 16
L = 16
NW = NC * NS
BPW = B // NW
NFULL = N_INPUT // L
TAIL_OFF = N_INPUT - L


def _prologue_body(m_ref, x_ref, xo_ref):
    m2 = m_ref[...]
    cols = lax.broadcasted_iota(jnp.int32, (1, N_INPUT), 1)
    hit = jnp.any(m2 == cols, axis=0, keepdims=True)
    colmask = jnp.where(hit, 0.0, 1.0)
    xo_ref[...] = jnp.log1p(x_ref[...] * colmask)


HALF = N_HIDDEN // 2
NPAIR = BPW // 2


def _sc_matvec_body(x_hbm, m_hbm, tab_hbm, h_hbm,
                    m_v, idx0, idx1, x0, x1, a0, a1, h_v, hstage,
                    sem_a0, sem_a1, sem_x0, sem_x1):
    wid = lax.axis_index("s") * NC + lax.axis_index("c")
    base = wid * BPW
    pltpu.sync_copy(m_hbm.at[pl.ds(base, BPW)], m_v)
    lane = lax.iota(jnp.int32, L)
    tail_mask = (lane >= L - N_INPUT % L).astype(jnp.float32)

    def gene16(e):
        return plsc.load_gather(m_v, [jnp.full((L,), e, jnp.int32)])

    def build_idx(idx_ref, m16, half):
        row0 = m16 * N_HIDDEN + half * HALF
        for c in range(HALF // L):
            idx_ref[pl.ds(L * c, L)] = row0 + (L * c) + lane

    def start_a(idx_ref, a_ref, sem):
        pltpu.async_copy(tab_hbm.at[idx_ref], a_ref, sem)

    def drain_a(idx_ref, a_ref, sem):
        pltpu.make_async_copy(tab_hbm.at[idx_ref], a_ref, sem).wait()

    def compute_half(a_ref, x_ref, e, half):
        for g in range(HALF // L):
            def kbody(k, accs):
                off = pl.multiple_of(L * k, L)
                xk = x_ref[pl.ds(off, L)]
                return tuple(accs[j] + a_ref[L * g + j, pl.ds(off, L)] * xk
                             for j in range(L))
            accs = lax.fori_loop(
                0, NFULL, kbody,
                tuple(jnp.zeros((L,), jnp.float32) for _ in range(L)))
            xt = x_ref[pl.ds(TAIL_OFF, L)] * tail_mask
            for j in range(L):
                hstage[pl.ds(L * j, L)] = (
                    accs[j] + a_ref[L * g + j, pl.ds(TAIL_OFF, L)] * xt)
            hvec = jnp.zeros((L,), jnp.float32)
            for c in range(L):
                hvec = hvec + plsc.load_gather(hstage, [lane * L + c])
            off_h = pl.multiple_of(e * N_HIDDEN + half * HALF + L * g, L)
            h_v[pl.ds(off_h, L)] = hvec

    def pair_body(i, carry):
        e0 = 2 * i
        e1 = 2 * i + 1
        cpx0 = pltpu.async_copy(x_hbm.at[base + e0], x0, sem_x0)
        cpx1 = pltpu.async_copy(x_hbm.at[base + e1], x1, sem_x1)
        m16_e0 = gene16(e0)

        @pl.when(i == 0)
        def _first_fetch():
            build_idx(idx0, m16_e0, 0)
            start_a(idx0, a0, sem_a0)

        build_idx(idx1, m16_e0, 1)
        start_a(idx1, a1, sem_a1)
        m16_e1 = gene16(e1)
        drain_a(idx0, a0, sem_a0)
        cpx0.wait()
        compute_half(a0, x0, e0, 0)
        build_idx(idx0, m16_e1, 0)
        start_a(idx0, a0, sem_a0)
        drain_a(idx1, a1, sem_a1)
        compute_half(a1, x0, e0, 1)
        build_idx(idx1, m16_e1, 1)
        start_a(idx1, a1, sem_a1)
        drain_a(idx0, a0, sem_a0)
        cpx1.wait()
        compute_half(a0, x1, e1, 0)

        @pl.when(i < NPAIR - 1)
        def _prefetch_next():
            build_idx(idx0, gene16(e1 + 1), 0)
            start_a(idx0, a0, sem_a0)

        drain_a(idx1, a1, sem_a1)
        compute_half(a1, x1, e1, 1)
        return carry

    lax.fori_loop(0, NPAIR, pair_body, 0)
    pltpu.sync_copy(h_v, h_hbm.at[pl.ds(base * N_HIDDEN, BPW * N_HIDDEN)])


def _epilogue_body(h_ref, m_ref, bvt_ref, w1_ref, b1_ref, s_ref, lb_ref,
                   w3_ref, b3_ref, w4_ref, b4_ref, mean_ref, scale_ref):
    cols = lax.broadcasted_iota(jnp.int32, (B, N_INPUT), 1)
    onehot = (m_ref[...] == cols).astype(jnp.float32)
    h = h_ref[...] + jnp.dot(onehot, bvt_ref[...],
                             preferred_element_type=jnp.float32)
    h = jnp.dot(h, w1_ref[...], preferred_element_type=jnp.float32) + b1_ref[...]
    mu = jnp.mean(h, axis=-1, keepdims=True)
    var = jnp.mean((h - mu) ** 2, axis=-1, keepdims=True)
    h = (h - mu) * lax.rsqrt(var + 1e-6) * s_ref[...] + lb_ref[...]
    h = jnp.maximum(h, 0.0)
    mean_ref[...] = jnp.dot(h, w3_ref[...], preferred_element_type=jnp.float32) + b3_ref[...]
    scale_ref[...] = jnp.exp(
        jnp.dot(h, w4_ref[...], preferred_element_type=jnp.float32) + b4_ref[...])


def kernel(x, masked_genes, amats_table, bvecs_table, W1, b1, ln_scale, ln_bias,
           W3, b3, W4, b4):
    m = masked_genes.astype(jnp.int32)

    x_ = pl.pallas_call(
        _prologue_body,
        out_shape=jax.ShapeDtypeStruct((B, N_INPUT), jnp.float32),
    )(m.reshape(B, 1), x)

    tab2 = amats_table.reshape(N_INPUT * N_HIDDEN, N_INPUT)

    sc = pl.kernel(
        _sc_matvec_body,
        out_type=jax.ShapeDtypeStruct((B * N_HIDDEN,), jnp.float32),
        mesh=plsc.VectorSubcoreMesh(core_axis_name="c", subcore_axis_name="s"),
        compiler_params=pltpu.CompilerParams(needs_layout_passes=False,
                                             use_tc_tiling_on_sc=False),
        scratch_types=[
            pltpu.VMEM((BPW,), jnp.int32),
            pltpu.VMEM((HALF,), jnp.int32),
            pltpu.VMEM((HALF,), jnp.int32),
            pltpu.VMEM((N_INPUT,), jnp.float32),
            pltpu.VMEM((N_INPUT,), jnp.float32),
            pltpu.VMEM((HALF, N_INPUT), jnp.float32),
            pltpu.VMEM((HALF, N_INPUT), jnp.float32),
            pltpu.VMEM((BPW * N_HIDDEN,), jnp.float32),
            pltpu.VMEM((L * L,), jnp.float32),
            pltpu.SemaphoreType.DMA,
            pltpu.SemaphoreType.DMA,
            pltpu.SemaphoreType.DMA,
            pltpu.SemaphoreType.DMA,
        ],
    )
    h = sc(x_, m, tab2)

    mean, scale = pl.pallas_call(
        _epilogue_body,
        out_shape=(
            jax.ShapeDtypeStruct((B, N_LATENT), jnp.float32),
            jax.ShapeDtypeStruct((B, N_LATENT), jnp.float32),
        ),
    )(h.reshape(B, N_HIDDEN), m.reshape(B, 1), bvecs_table,
      W1, b1.reshape(1, N_HIDDEN),
      ln_scale.reshape(1, N_HIDDEN), ln_bias.reshape(1, N_HIDDEN),
      W3, b3.reshape(1, N_LATENT), W4, b4.reshape(1, N_LATENT))
    return (mean, scale)


# speedup vs baseline: 2.7147x; 1.0258x over previous
import functools

import jax
import jax.numpy as jnp
from jax import lax
from jax.experimental import pallas as pl
from jax.experimental.pallas import tpu as pltpu
from jax.experimental.pallas import tpu_sc as plsc

N_INPUT = 1000
N_HIDDEN = 64
N_LATENT = 32
B = 1024

NC = 2
NS = 16
L = 16
NW = NC * NS
BPW = B // NW
NFULL = N_INPUT // L
TAIL_OFF = N_INPUT - L


def _prologue_body(m_ref, x_ref, xo_ref):
    m2 = m_ref[...]
    cols = lax.broadcasted_iota(jnp.int32, (1, N_INPUT), 1)
    hit = jnp.any(m2 == cols, axis=0, keepdims=True)
    colmask = jnp.where(hit, 0.0, 1.0)
    xo_ref[...] = jnp.log1p(x_ref[...] * colmask)


NPAIR = BPW // 2
HFLUSH = 8


def _sc_matvec_body(x_hbm, m_hbm, tab_hbm, h_hbm,
                    m_v, idx0, idx1, x0, x1, a0, a1, h_8, hstage,
                    sem_a0, sem_a1, sem_x0, sem_x1):
    wid = lax.axis_index("s") * NC + lax.axis_index("c")
    base = wid * BPW
    pltpu.sync_copy(m_hbm.at[pl.ds(base, BPW)], m_v)
    lane = lax.iota(jnp.int32, L)
    tail_mask = (lane >= L - N_INPUT % L).astype(jnp.float32)

    def gene16(e):
        return plsc.load_gather(m_v, [jnp.full((L,), e, jnp.int32)])

    def start_a(idx_ref, m16, a_ref, sem):
        idx_ref[...] = m16
        pltpu.async_copy(tab_hbm.at[idx_ref.at[pl.ds(0, 1)]], a_ref, sem)

    def drain_a(idx_ref, a_ref, sem):
        pltpu.make_async_copy(tab_hbm.at[idx_ref.at[pl.ds(0, 1)]], a_ref,
                              sem).wait()

    def compute_example(a_ref, x_ref, e):
        for g in range(N_HIDDEN // L):
            def kbody(k, accs):
                off = pl.multiple_of(L * k, L)
                xk = x_ref[pl.ds(off, L)]
                return tuple(
                    accs[j]
                    + a_ref[0, pl.ds((L * g + j) * N_INPUT + off, L)] * xk
                    for j in range(L))
            accs = lax.fori_loop(
                0, NFULL, kbody,
                tuple(jnp.zeros((L,), jnp.float32) for _ in range(L)))
            xt = x_ref[pl.ds(TAIL_OFF, L)] * tail_mask
            for j in range(L):
                hstage[pl.ds(L * j, L)] = (
                    accs[j]
                    + a_ref[0, pl.ds((L * g + j) * N_INPUT + TAIL_OFF, L)] * xt)
            hvec = jnp.zeros((L,), jnp.float32)
            for c in range(L):
                hvec = hvec + plsc.load_gather(hstage, [lane * L + c])
            off_h = pl.multiple_of((e % HFLUSH) * N_HIDDEN + L * g, L)
            h_8[pl.ds(off_h, L)] = hvec

    def pair_body(i, carry):
        e0 = 2 * i
        e1 = 2 * i + 1
        cpx0 = pltpu.async_copy(x_hbm.at[base + e0], x0, sem_x0)
        cpx1 = pltpu.async_copy(x_hbm.at[base + e1], x1, sem_x1)
        m16_e0 = gene16(e0)
        m16_e1 = gene16(e1)

        @pl.when(i == 0)
        def _first_fetch():
            start_a(idx0, m16_e0, a0, sem_a0)

        start_a(idx1, m16_e1, a1, sem_a1)
        drain_a(idx0, a0, sem_a0)
        cpx0.wait()
        compute_example(a0, x0, e0)

        @pl.when(i < NPAIR - 1)
        def _prefetch_next():
            start_a(idx0, gene16(e0 + 2), a0, sem_a0)

        drain_a(idx1, a1, sem_a1)
        cpx1.wait()
        compute_example(a1, x1, e1)

        @pl.when(i % (HFLUSH // 2) == (HFLUSH // 2) - 1)
        def _flush_h():
            q = i // (HFLUSH // 2)
            pltpu.sync_copy(
                h_8,
                h_hbm.at[pl.ds((base + HFLUSH * q) * N_HIDDEN,
                               HFLUSH * N_HIDDEN)])

        return carry

    lax.fori_loop(0, NPAIR, pair_body, 0)


def _epilogue_body(h_ref, m_ref, bvt_ref, w1_ref, b1_ref, s_ref, lb_ref,
                   w3_ref, b3_ref, w4_ref, b4_ref, mean_ref, scale_ref):
    cols = lax.broadcasted_iota(jnp.int32, (B, N_INPUT), 1)
    onehot = (m_ref[...] == cols).astype(jnp.float32)
    h = h_ref[...] + jnp.dot(onehot, bvt_ref[...],
                             preferred_element_type=jnp.float32)
    h = jnp.dot(h, w1_ref[...], preferred_element_type=jnp.float32) + b1_ref[...]
    mu = jnp.mean(h, axis=-1, keepdims=True)
    var = jnp.mean((h - mu) ** 2, axis=-1, keepdims=True)
    h = (h - mu) * lax.rsqrt(var + 1e-6) * s_ref[...] + lb_ref[...]
    h = jnp.maximum(h, 0.0)
    mean_ref[...] = jnp.dot(h, w3_ref[...], preferred_element_type=jnp.float32) + b3_ref[...]
    scale_ref[...] = jnp.exp(
        jnp.dot(h, w4_ref[...], preferred_element_type=jnp.float32) + b4_ref[...])


def kernel(x, masked_genes, amats_table, bvecs_table, W1, b1, ln_scale, ln_bias,
           W3, b3, W4, b4):
    m = masked_genes.astype(jnp.int32)

    x_ = pl.pallas_call(
        _prologue_body,
        out_shape=jax.ShapeDtypeStruct((B, N_INPUT), jnp.float32),
    )(m.reshape(B, 1), x)

    sc = pl.kernel(
        _sc_matvec_body,
        out_type=jax.ShapeDtypeStruct((B * N_HIDDEN,), jnp.float32),
        mesh=plsc.VectorSubcoreMesh(core_axis_name="c", subcore_axis_name="s"),
        compiler_params=pltpu.CompilerParams(needs_layout_passes=False,
                                             use_tc_tiling_on_sc=False),
        scratch_types=[
            pltpu.VMEM((BPW,), jnp.int32),
            pltpu.VMEM((L,), jnp.int32),
            pltpu.VMEM((L,), jnp.int32),
            pltpu.VMEM((N_INPUT,), jnp.float32),
            pltpu.VMEM((N_INPUT,), jnp.float32),
            pltpu.VMEM((1, N_HIDDEN * N_INPUT), jnp.float32),
            pltpu.VMEM((1, N_HIDDEN * N_INPUT), jnp.float32),
            pltpu.VMEM((HFLUSH * N_HIDDEN,), jnp.float32),
            pltpu.VMEM((L * L,), jnp.float32),
            pltpu.SemaphoreType.DMA,
            pltpu.SemaphoreType.DMA,
            pltpu.SemaphoreType.DMA,
            pltpu.SemaphoreType.DMA,
        ],
    )
    h = sc(x_, m, amats_table)

    mean, scale = pl.pallas_call(
        _epilogue_body,
        out_shape=(
            jax.ShapeDtypeStruct((B, N_LATENT), jnp.float32),
            jax.ShapeDtypeStruct((B, N_LATENT), jnp.float32),
        ),
    )(h.reshape(B, N_HIDDEN), m.reshape(B, 1), bvecs_table,
      W1, b1.reshape(1, N_HIDDEN),
      ln_scale.reshape(1, N_HIDDEN), ln_bias.reshape(1, N_HIDDEN),
      W3, b3.reshape(1, N_LATENT), W4, b4.reshape(1, N_LATENT))
    return (mean, scale)


# speedup vs baseline: 5.9856x; 2.2049x over previous
import functools

import jax
import jax.numpy as jnp
from jax import lax
from jax.experimental import pallas as pl
from jax.experimental.pallas import tpu as pltpu
from jax.experimental.pallas import tpu_sc as plsc

N_INPUT = 1000
N_HIDDEN = 64
N_LATENT = 32
B = 1024

NC = 2
NS = 16
L = 16
NW = NC * NS
BPW = B // NW
NFULL = N_INPUT // L
TAIL_OFF = N_INPUT - L


XPAD = 1024


def _prologue_body(m_ref, x_ref, xo_ref):
    m2 = m_ref[...]
    cols = lax.broadcasted_iota(jnp.int32, (1, N_INPUT), 1)
    hit = jnp.any(m2 == cols, axis=0, keepdims=True)
    colmask = jnp.where(hit, 0.0, 1.0)
    xo_ref[:, :N_INPUT] = jnp.log1p(x_ref[...] * colmask)
    xo_ref[:, N_INPUT:] = jnp.zeros((B, XPAD - N_INPUT), jnp.float32)


NPAIR = BPW // 2
HFLUSH = 4


def _sc_matvec_body(x_hbm, m_hbm, tab_hbm, h_hbm,
                    m_v, idx_b, x0, x1, a0, a1, h_8, hstage,
                    sem_a0, sem_a1, sem_x0, sem_x1):
    wid = lax.axis_index("s") * NC + lax.axis_index("c")
    pltpu.sync_copy(m_hbm.at[pl.ds((wid // 4) * 128, 128)], m_v)
    mbase = (wid % 4) * BPW
    base = wid * BPW
    lane = lax.iota(jnp.int32, L)
    tail_mask = (lane >= L - N_INPUT % L).astype(jnp.float32)

    def gene16(e):
        return plsc.load_gather(m_v, [jnp.full((L,), mbase + e, jnp.int32)])

    def start_a(slot, m16, a_ref, sem):
        idx_b[pl.ds(L * slot, L)] = m16
        pltpu.async_copy(tab_hbm.at[idx_b.at[pl.ds(L * slot, 1)]], a_ref, sem)

    def drain_a(slot, a_ref, sem):
        pltpu.make_async_copy(tab_hbm.at[idx_b.at[pl.ds(L * slot, 1)]], a_ref,
                              sem).wait()

    def compute_example(a_ref, x_ref, e):
        for g in range(N_HIDDEN // L):
            def kbody(k, accs):
                off = pl.multiple_of(L * k, L)
                xk = x_ref[pl.ds(off, L)]
                return tuple(
                    accs[j]
                    + a_ref[0, pl.ds((L * g + j) * N_INPUT + off, L)] * xk
                    for j in range(L))
            accs = lax.fori_loop(
                0, NFULL, kbody,
                tuple(jnp.zeros((L,), jnp.float32) for _ in range(L)))
            xt = x_ref[pl.ds(TAIL_OFF, L)] * tail_mask
            for j in range(L):
                hstage[pl.ds(L * j, L)] = (
                    accs[j]
                    + a_ref[0, pl.ds((L * g + j) * N_INPUT + TAIL_OFF, L)] * xt)
            hvec = jnp.zeros((L,), jnp.float32)
            for c in range(L):
                hvec = hvec + plsc.load_gather(hstage, [lane * L + c])
            off_h = pl.multiple_of((e % HFLUSH) * N_HIDDEN + L * g, L)
            h_8[pl.ds(off_h, L)] = hvec

    def pair_body(i, carry):
        e0 = 2 * i
        e1 = 2 * i + 1
        cpx0 = pltpu.async_copy(x_hbm.at[base + e0], x0, sem_x0)
        cpx1 = pltpu.async_copy(x_hbm.at[base + e1], x1, sem_x1)
        m16_e0 = gene16(e0)
        m16_e1 = gene16(e1)

        @pl.when(i == 0)
        def _first_fetch():
            start_a(0, m16_e0, a0, sem_a0)

        start_a(1, m16_e1, a1, sem_a1)
        drain_a(0, a0, sem_a0)
        cpx0.wait()
        compute_example(a0, x0, e0)

        @pl.when(i < NPAIR - 1)
        def _prefetch_next():
            start_a(0, gene16(e0 + 2), a0, sem_a0)

        drain_a(1, a1, sem_a1)
        cpx1.wait()
        compute_example(a1, x1, e1)

        @pl.when(i % (HFLUSH // 2) == (HFLUSH // 2) - 1)
        def _flush_h():
            q = i // (HFLUSH // 2)
            pltpu.sync_copy(
                h_8,
                h_hbm.at[pl.ds((base + HFLUSH * q) * N_HIDDEN,
                               HFLUSH * N_HIDDEN)])

        return carry

    lax.fori_loop(0, NPAIR, pair_body, 0)


def _epilogue_body(h_ref, m_ref, bvt_ref, w1_ref, b1_ref, s_ref, lb_ref,
                   w3_ref, b3_ref, w4_ref, b4_ref, mean_ref, scale_ref):
    cols = lax.broadcasted_iota(jnp.int32, (B, N_INPUT), 1)
    onehot = (m_ref[...] == cols).astype(jnp.float32)
    h = h_ref[...] + jnp.dot(onehot, bvt_ref[...],
                             preferred_element_type=jnp.float32)
    h = jnp.dot(h, w1_ref[...], preferred_element_type=jnp.float32) + b1_ref[...]
    mu = jnp.mean(h, axis=-1, keepdims=True)
    var = jnp.mean((h - mu) ** 2, axis=-1, keepdims=True)
    h = (h - mu) * lax.rsqrt(var + 1e-6) * s_ref[...] + lb_ref[...]
    h = jnp.maximum(h, 0.0)
    mean_ref[...] = jnp.dot(h, w3_ref[...], preferred_element_type=jnp.float32) + b3_ref[...]
    scale_ref[...] = jnp.exp(
        jnp.dot(h, w4_ref[...], preferred_element_type=jnp.float32) + b4_ref[...])


def kernel(x, masked_genes, amats_table, bvecs_table, W1, b1, ln_scale, ln_bias,
           W3, b3, W4, b4):
    m = masked_genes.astype(jnp.int32)

    x_ = pl.pallas_call(
        _prologue_body,
        out_shape=jax.ShapeDtypeStruct((B, XPAD), jnp.float32),
    )(m.reshape(B, 1), x)

    sc = pl.kernel(
        _sc_matvec_body,
        out_type=jax.ShapeDtypeStruct((B * N_HIDDEN,), jnp.float32),
        mesh=plsc.VectorSubcoreMesh(core_axis_name="c", subcore_axis_name="s"),
        compiler_params=pltpu.CompilerParams(needs_layout_passes=False,
                                             use_tc_tiling_on_sc=True),
        scratch_types=[
            pltpu.VMEM((128,), jnp.int32),
            pltpu.VMEM((2 * L,), jnp.int32),
            pltpu.VMEM((XPAD,), jnp.float32),
            pltpu.VMEM((XPAD,), jnp.float32),
            pltpu.VMEM((1, N_HIDDEN * N_INPUT), jnp.float32),
            pltpu.VMEM((1, N_HIDDEN * N_INPUT), jnp.float32),
            pltpu.VMEM((HFLUSH * N_HIDDEN,), jnp.float32),
            pltpu.VMEM((L * L,), jnp.float32),
            pltpu.SemaphoreType.DMA,
            pltpu.SemaphoreType.DMA,
            pltpu.SemaphoreType.DMA,
            pltpu.SemaphoreType.DMA,
        ],
    )
    h = sc(x_, m, amats_table)

    mean, scale = pl.pallas_call(
        _epilogue_body,
        out_shape=(
            jax.ShapeDtypeStruct((B, N_LATENT), jnp.float32),
            jax.ShapeDtypeStruct((B, N_LATENT), jnp.float32),
        ),
    )(h.reshape(B, N_HIDDEN), m.reshape(B, 1), bvecs_table,
      W1, b1.reshape(1, N_HIDDEN),
      ln_scale.reshape(1, N_HIDDEN), ln_bias.reshape(1, N_HIDDEN),
      W3, b3.reshape(1, N_LATENT), W4, b4.reshape(1, N_LATENT))
    return (mean, scale)
